# baseline (speedup 1.0000x reference)
import functools

import numpy as np
import jax
import jax.numpy as jnp
from jax import lax
from jax.experimental import pallas as pl
from jax.experimental.pallas import tpu as pltpu
from jax.experimental.pallas import tpu_sc as plsc

B = 16384
K = 50
NUM_ATOMS = 1000000

NW = 32
R = B // NW
NQ = 4
Q = R // NQ
EQ = Q * K


def _finish_body(h_ref, rns_ref, rowsum_ref, out_ref):
    s = jnp.sum(h_ref[:])
    novelty = jnp.sqrt(jnp.maximum(rns_ref[:], 0.0))
    out_ref[:] = rowsum_ref[:] * novelty / jnp.maximum(s, 1e-9)


def _tc_finish(h, rns, rowsum):
    out = pl.pallas_call(
        _finish_body,
        out_shape=jax.ShapeDtypeStruct((128, 128), jnp.float32),
    )(h.reshape(1000, 1000), rns.reshape(128, 128), rowsum.reshape(128, 128))
    return out.reshape(B)


_mesh = plsc.VectorSubcoreMesh(core_axis_name="c", subcore_axis_name="s")


@functools.partial(
    pl.kernel,
    mesh=_mesh,
    out_type=jax.ShapeDtypeStruct((B,), jnp.float32),
    compiler_params=pltpu.CompilerParams(needs_layout_passes=False),
    scratch_types=[
        pltpu.VMEM((EQ,), jnp.int32),
        pltpu.VMEM((EQ,), jnp.int32),
        pltpu.VMEM((EQ,), jnp.float32),
        pltpu.VMEM((EQ,), jnp.float32),
        pltpu.VMEM((EQ,), jnp.float32),
        pltpu.VMEM((EQ,), jnp.float32),
        pltpu.VMEM((R,), jnp.float32),
        pltpu.SemaphoreType.DMA,
        pltpu.SemaphoreType.DMA,
        pltpu.SemaphoreType.DMA,
        pltpu.SemaphoreType.DMA,
        pltpu.SemaphoreType.DMA,
        pltpu.SemaphoreType.DMA,
    ],
)
def _sc_gather_reduce(idxf_hbm, valsf_hbm, table_hbm, out_hbm,
                      idxf_a, idxf_b, valsf_a, valsf_b, hf_a, hf_b, out_v,
                      sem_i0, sem_i1, sem_v0, sem_v1, sem_g0, sem_g1):
    wid = lax.axis_index("s") * 2 + lax.axis_index("c")
    base_e = wid * R * K

    idxf_bufs = [idxf_a, idxf_b]
    valsf_bufs = [valsf_a, valsf_b]
    hf_bufs = [hf_a, hf_b]
    sem_i = [sem_i0, sem_i1]
    sem_v = [sem_v0, sem_v1]
    sem_g = [sem_g0, sem_g1]

    def copy_idx(q, p):
        return pltpu.async_copy(
            idxf_hbm.at[pl.ds(base_e + q * EQ, EQ)], idxf_bufs[p], sem_i[p])

    def copy_vals(q, p):
        return pltpu.async_copy(
            valsf_hbm.at[pl.ds(base_e + q * EQ, EQ)], valsf_bufs[p], sem_v[p])

    def fire_gather(p):
        return pltpu.async_copy(
            table_hbm.at[idxf_bufs[p].at[:]], hf_bufs[p], sem_g[p])

    def compute_quarter(q, p):
        valsf = valsf_bufs[p]
        hf = hf_bufs[p]
        def g_body(g, _):
            fbase = g * 16 * K
            acc = jnp.zeros((16,), jnp.float32)
            for k in range(K):
                iv = lax.iota(jnp.int32, 16) * K + (fbase + k)
                h16 = plsc.load_gather(hf, [iv])
                v16 = plsc.load_gather(valsf, [iv])
                acc = acc + jnp.maximum(v16, 0.0) * h16
            out_v[pl.ds(q * Q + g * 16, 16)] = acc
            return 0
        lax.fori_loop(0, Q // 16, g_body, 0)

    cis = [None, None]
    cvs = [None, None]
    gathers = [None, None]
    cis[0] = copy_idx(0, 0)
    cvs[0] = copy_vals(0, 0)
    cis[0].wait()
    gathers[0] = fire_gather(0)
    cis[1] = copy_idx(1, 1)
    cvs[1] = copy_vals(1, 1)
    for q in range(NQ):
        p = q & 1
        if q + 1 < NQ:
            cis[p ^ 1].wait()
            gathers[p ^ 1] = fire_gather(p ^ 1)
        cvs[p].wait()
        gathers[p].wait()
        if q + 2 < NQ:
            cis[p] = copy_idx(q + 2, p)
        compute_quarter(q, p)
        if q + 2 < NQ:
            cvs[p] = copy_vals(q + 2, p)

    pltpu.sync_copy(out_v, out_hbm.at[pl.ds(wid * R, R)])


def kernel(residual_norm_sq, topk_idx, topk_vals, atom_hazard_prior):
    idx_flat = topk_idx.astype(jnp.int32).reshape(B * K)
    vals_flat = topk_vals.reshape(B * K)
    rowsum = _sc_gather_reduce(idx_flat, vals_flat, atom_hazard_prior)
    return _tc_finish(atom_hazard_prior, residual_norm_sq, rowsum)


# speedup vs baseline: 1.2984x; 1.2984x over previous
import functools

import numpy as np
import jax
import jax.numpy as jnp
from jax import lax
from jax.experimental import pallas as pl
from jax.experimental.pallas import tpu as pltpu
from jax.experimental.pallas import tpu_sc as plsc

B = 16384
K = 50
NUM_ATOMS = 1000000

NW = 32
R = B // NW
NQ = 4
Q = R // NQ
EQ = Q * K
G = Q // 16


def _finish_body(h_ref, rns_ref, rowsum_ref, out_ref):
    s = jnp.sum(h_ref[:])
    novelty = jnp.sqrt(jnp.maximum(rns_ref[:], 0.0))
    out_ref[:] = rowsum_ref[:] * novelty / jnp.maximum(s, 1e-9)


def _tc_finish(h, rns, rowsum):
    out = pl.pallas_call(
        _finish_body,
        out_shape=jax.ShapeDtypeStruct((128, 128), jnp.float32),
    )(h.reshape(1000, 1000), rns.reshape(128, 128), rowsum.reshape(128, 128))
    return out.reshape(B)


_mesh = plsc.VectorSubcoreMesh(core_axis_name="c", subcore_axis_name="s")


@functools.partial(
    pl.kernel,
    mesh=_mesh,
    out_type=jax.ShapeDtypeStruct((B,), jnp.float32),
    compiler_params=pltpu.CompilerParams(needs_layout_passes=False),
    scratch_types=[
        pltpu.VMEM((EQ,), jnp.int32),
        pltpu.VMEM((EQ,), jnp.int32),
        pltpu.VMEM((EQ,), jnp.float32),
        pltpu.VMEM((EQ,), jnp.float32),
        pltpu.VMEM((EQ,), jnp.float32),
        pltpu.VMEM((EQ,), jnp.float32),
        pltpu.VMEM((R,), jnp.float32),
        pltpu.SemaphoreType.DMA,
        pltpu.SemaphoreType.DMA,
        pltpu.SemaphoreType.DMA,
        pltpu.SemaphoreType.DMA,
        pltpu.SemaphoreType.DMA,
        pltpu.SemaphoreType.DMA,
    ],
)
def _sc_gather_reduce(idx_hbm, vals_hbm, table_hbm, out_hbm,
                      idx_a, idx_b, vals_a, vals_b, hf_a, hf_b, out_v,
                      sem_i0, sem_i1, sem_v0, sem_v1, sem_g0, sem_g1):
    wid = lax.axis_index("s") * 2 + lax.axis_index("c")
    base_e = wid * R * K

    idx_bufs = [idx_a, idx_b]
    vals_bufs = [vals_a, vals_b]
    hf_bufs = [hf_a, hf_b]
    sem_i = [sem_i0, sem_i1]
    sem_v = [sem_v0, sem_v1]
    sem_g = [sem_g0, sem_g1]

    def copy_idx(q, p):
        return pltpu.async_copy(
            idx_hbm.at[pl.ds(base_e + q * EQ, EQ)], idx_bufs[p], sem_i[p])

    def copy_vals(q, p):
        return pltpu.async_copy(
            vals_hbm.at[pl.ds(base_e + q * EQ, EQ)], vals_bufs[p], sem_v[p])

    def fire_gather(p):
        return pltpu.async_copy(
            table_hbm.at[idx_bufs[p].at[:]], hf_bufs[p], sem_g[p])

    def compute_quarter(q, p):
        vf = vals_bufs[p]
        hf = hf_bufs[p]
        def g_body(g, _):
            acc = jnp.zeros((16,), jnp.float32)
            for k in range(K):
                off = k * Q + g * 16
                h16 = hf[pl.ds(off, 16)]
                v16 = vf[pl.ds(off, 16)]
                acc = acc + jnp.maximum(v16, 0.0) * h16
            out_v[pl.ds(q * Q + g * 16, 16)] = acc
            return 0
        lax.fori_loop(0, G, g_body, 0)

    cis = [None, None]
    cvs = [None, None]
    gathers = [None, None]
    cis[0] = copy_idx(0, 0)
    cvs[0] = copy_vals(0, 0)
    cis[0].wait()
    gathers[0] = fire_gather(0)
    cis[1] = copy_idx(1, 1)
    cvs[1] = copy_vals(1, 1)
    for q in range(NQ):
        p = q & 1
        if q + 1 < NQ:
            cis[p ^ 1].wait()
            gathers[p ^ 1] = fire_gather(p ^ 1)
        cvs[p].wait()
        gathers[p].wait()
        if q + 2 < NQ:
            cis[p] = copy_idx(q + 2, p)
        compute_quarter(q, p)
        if q + 2 < NQ:
            cvs[p] = copy_vals(q + 2, p)

    pltpu.sync_copy(out_v, out_hbm.at[pl.ds(wid * R, R)])


def _blocked_transpose(x):
    return x.reshape(NW * NQ, Q, K).transpose(0, 2, 1).reshape(B * K)


def kernel(residual_norm_sq, topk_idx, topk_vals, atom_hazard_prior):
    idx_t = _blocked_transpose(topk_idx.astype(jnp.int32))
    vals_t = _blocked_transpose(topk_vals)
    rowsum = _sc_gather_reduce(idx_t, vals_t, atom_hazard_prior)
    return _tc_finish(atom_hazard_prior, residual_norm_sq, rowsum)
